# baseline (speedup 1.0000x reference)
import jax
import jax.numpy as jnp
from jax import lax
from jax.experimental import pallas as pl
from jax.experimental.pallas import tpu as pltpu
from jax.experimental.pallas import tpu_sc as plsc

N_USERS = 4096
N_ITEMS = 1000
H = 12
A = 1 << H
R = 256
NW = 32
UPW = N_USERS // NW
G = 16
NG = UPW // G
WORDS = N_ITEMS // 4


def _eb_body(dist_ref, eb_ref):
    dist = dist_ref[...]
    k_ids = lax.broadcasted_iota(jnp.int32, (A, 128), 0)
    j_ids = jnp.minimum(lax.broadcasted_iota(jnp.int32, (A, 128), 1), 31)
    bitmat = ((k_ids >> j_ids) & 1).astype(jnp.float32)
    eb = jnp.dot(dist, bitmat, preferred_element_type=jnp.float32)
    eb_ref[...] = eb[:, :16]


def _sc_body(eb_hbm, hist_hbm, o_hbm, act_hbm, num_hbm,
             eb_v, hist_v, obuf, wbuf, num_v):
    wid = lax.axis_index("s") * 2 + lax.axis_index("c")
    base = wid * UPW
    pltpu.sync_copy(eb_hbm.at[pl.ds(base, UPW), :], eb_v)
    pltpu.sync_copy(hist_hbm.at[pl.ds(base, UPW), :], hist_v)

    zf = jnp.zeros((G,), jnp.float32)
    zi = jnp.zeros((G,), jnp.int32)

    def zero_o(i, carry):
        obuf[pl.ds(i * 16, 16)] = zf
        return carry

    lax.fori_loop(0, G * N_ITEMS // 16, zero_o, 0)

    def zero_w(i, carry):
        wbuf[pl.ds(i * 16, 16)] = zi
        return carry

    lax.fori_loop(0, G * WORDS // 16, zero_w, 0)

    rows = lax.broadcasted_iota(jnp.int32, (G,), 0)
    row_o = rows * N_ITEMS
    row_w = rows * WORDS

    def group(g, carry):
        urows = g * G + rows

        def hcol(j):
            return plsc.load_gather(hist_v, [urows, jnp.full((G,), j, jnp.int32)])

        for j in range(H):
            col = hcol(j)
            val = plsc.load_gather(eb_v, [urows, jnp.full((G,), j, jnp.int32)])
            plsc.addupdate_scatter(obuf, [row_o + col], val)
        for j in range(H):
            col = hcol(j)
            oval = plsc.load_gather(obuf, [row_o + col])
            bit = (oval > 0.5).astype(jnp.int32)
            widx = row_w + (col >> 2)
            sh = (col & 3) * 8
            wold = plsc.load_gather(wbuf, [widx])
            wnew = (wold & jnp.bitwise_not(jnp.left_shift(jnp.int32(255), sh))) \
                | (bit << sh)
            plsc.store_scatter(wbuf, [widx], wnew)
        num = jnp.zeros((G,), jnp.int32)
        for c in range(H):
            oval = plsc.load_gather(obuf, [row_o + c])
            num = num | ((oval > 0.5).astype(jnp.int32) << c)
        num_v[pl.ds(g * G, G)] = num
        u0 = base + g * G
        pltpu.sync_copy(obuf, o_hbm.at[pl.ds(u0 * N_ITEMS, G * N_ITEMS)])
        pltpu.sync_copy(wbuf, act_hbm.at[pl.ds(u0 * WORDS, G * WORDS)])
        for j in range(H):
            col = hcol(j)
            plsc.store_scatter(obuf, [row_o + col], zf)
            plsc.store_scatter(wbuf, [row_w + (col >> 2)], zi)
        return carry

    lax.fori_loop(0, NG, group, 0)
    pltpu.sync_copy(num_v, num_hbm.at[pl.ds(base, UPW)])


def _make_sc_call(interpret=False):
    mesh = plsc.VectorSubcoreMesh(
        core_axis_name="c", subcore_axis_name="s", num_cores=2, num_subcores=16
    )
    return pl.kernel(
        _sc_body,
        out_type=[
            jax.ShapeDtypeStruct((N_USERS * N_ITEMS,), jnp.float32),
            jax.ShapeDtypeStruct((N_USERS * WORDS,), jnp.int32),
            jax.ShapeDtypeStruct((N_USERS,), jnp.int32),
        ],
        mesh=mesh,
        scratch_types=[
            pltpu.VMEM((UPW, 16), jnp.float32),
            pltpu.VMEM((UPW, H), jnp.int32),
            pltpu.VMEM((G * N_ITEMS,), jnp.float32),
            pltpu.VMEM((G * WORDS,), jnp.int32),
            pltpu.VMEM((UPW,), jnp.int32),
        ],
        compiler_params=pltpu.CompilerParams(needs_layout_passes=False),
        interpret=interpret,
    )


def kernel(distribution, history):
    hist = history.astype(jnp.int32)
    eb = pl.pallas_call(
        _eb_body,
        grid=(N_USERS // R,),
        in_specs=[pl.BlockSpec((R, A), lambda i: (i, 0))],
        out_specs=pl.BlockSpec((R, 16), lambda i: (i, 0)),
        out_shape=jax.ShapeDtypeStruct((N_USERS, 16), jnp.float32),
    )(distribution)
    o_flat, act_words, num = _make_sc_call()(eb, hist)
    o = o_flat.reshape(N_USERS, N_ITEMS)
    act = (
        lax.bitcast_convert_type(act_words, jnp.uint8)
        .reshape(N_USERS, N_ITEMS)
        .astype(jnp.bool_)
    )
    return (o, act, num)


# speedup vs baseline: 1.3357x; 1.3357x over previous
import jax
import jax.numpy as jnp
from jax import lax
from jax.experimental import pallas as pl
from jax.experimental.pallas import tpu as pltpu
from jax.experimental.pallas import tpu_sc as plsc

N_USERS = 4096
N_ITEMS = 1000
H = 12
A = 1 << H
R = 256
NW = 32
UPW = N_USERS // NW
G = 16
NG = UPW // G


def _eb_body(dist_ref, eb_ref):
    dist = dist_ref[...]
    k_ids = lax.broadcasted_iota(jnp.int32, (A, 128), 0)
    j_ids = jnp.minimum(lax.broadcasted_iota(jnp.int32, (A, 128), 1), 31)
    bitmat = ((k_ids >> j_ids) & 1).astype(jnp.float32)
    eb = jnp.dot(dist, bitmat, preferred_element_type=jnp.float32)
    eb_ref[...] = eb[:, :16]


def _sc_body(eb_hbm, hist_hbm, o_hbm, eb_v, hist_v, obuf):
    wid = lax.axis_index("s") * 2 + lax.axis_index("c")
    base = wid * UPW
    pltpu.sync_copy(eb_hbm.at[pl.ds(base, UPW), :], eb_v)
    pltpu.sync_copy(hist_hbm.at[pl.ds(base, UPW), :], hist_v)

    zf = jnp.zeros((G,), jnp.float32)
    rows = lax.broadcasted_iota(jnp.int32, (G,), 0)

    def zero_row(u, carry):
        def zero_chunk(i, c2):
            obuf[u, pl.ds(jnp.minimum(i * 16, N_ITEMS - 16), 16)] = zf
            return c2
        return lax.fori_loop(0, N_ITEMS // 16 + 1, zero_chunk, carry)

    lax.fori_loop(0, G, zero_row, 0)

    def group(g, carry):
        urows = g * G + rows

        def hcol(j):
            return plsc.load_gather(hist_v, [urows, jnp.full((G,), j, jnp.int32)])

        for j in range(H):
            col = hcol(j)
            val = plsc.load_gather(eb_v, [urows, jnp.full((G,), j, jnp.int32)])
            plsc.addupdate_scatter(obuf, [rows, col], val)
        pltpu.sync_copy(obuf, o_hbm.at[pl.ds(base + g * G, G), :])
        for j in range(H):
            plsc.store_scatter(obuf, [rows, hcol(j)], zf)
        return carry

    lax.fori_loop(0, NG, group, 0)


def _act_body(o_ref, act_ref, num_ref):
    o = o_ref[...]
    act = o > 0.5
    act_ref[...] = act.astype(jnp.int8)
    pw = (1 << lax.broadcasted_iota(jnp.int32, (R, H), 1)).astype(jnp.int32)
    num_ref[...] = jnp.sum(act[:, :H].astype(jnp.int32) * pw, axis=1, keepdims=True)


def _make_sc_call(interpret=False):
    mesh = plsc.VectorSubcoreMesh(
        core_axis_name="c", subcore_axis_name="s", num_cores=2, num_subcores=16
    )
    return pl.kernel(
        _sc_body,
        out_type=jax.ShapeDtypeStruct((N_USERS, N_ITEMS), jnp.float32),
        mesh=mesh,
        scratch_types=[
            pltpu.VMEM((UPW, 16), jnp.float32),
            pltpu.VMEM((UPW, H), jnp.int32),
            pltpu.VMEM((G, N_ITEMS), jnp.float32),
        ],
        compiler_params=pltpu.CompilerParams(needs_layout_passes=False),
        interpret=interpret,
    )


def kernel(distribution, history):
    hist = history.astype(jnp.int32)
    eb = pl.pallas_call(
        _eb_body,
        grid=(N_USERS // R,),
        in_specs=[pl.BlockSpec((R, A), lambda i: (i, 0))],
        out_specs=pl.BlockSpec((R, 16), lambda i: (i, 0)),
        out_shape=jax.ShapeDtypeStruct((N_USERS, 16), jnp.float32),
    )(distribution)
    o = _make_sc_call()(eb, hist)
    act8, num = pl.pallas_call(
        _act_body,
        grid=(N_USERS // R,),
        in_specs=[pl.BlockSpec((R, N_ITEMS), lambda i: (i, 0))],
        out_specs=[
            pl.BlockSpec((R, N_ITEMS), lambda i: (i, 0)),
            pl.BlockSpec((R, 1), lambda i: (i, 0)),
        ],
        out_shape=[
            jax.ShapeDtypeStruct((N_USERS, N_ITEMS), jnp.int8),
            jax.ShapeDtypeStruct((N_USERS, 1), jnp.int32),
        ],
    )(o)
    return (o, act8.astype(jnp.bool_), num.reshape(N_USERS))
